# baseline (speedup 1.0000x reference)
import functools

import jax
import jax.numpy as jnp
import numpy as np
from jax.experimental import pallas as pl

B = 4; T = 2048; D = 512; NH = 8; HD = D // NH
K = 1024; L = 4; WIN = 25; STRIDE = 12; HID = 128
NW = (T - WIN) // STRIDE + 1
NWP = 176
TKP = 256
BNWP = B * NWP
_H = jax.lax.Precision.HIGHEST
_F = jnp.float32


def _dot(a, b, dims=None):
    if dims is None:
        return jnp.dot(a, b, preferred_element_type=_F, precision=_H)
    return jax.lax.dot_general(a, b, (dims, ((), ())),
                               preferred_element_type=_F, precision=_H)


def _dot16(a, b, dims=((1,), (1,))):
    return jax.lax.dot_general(a.astype(jnp.bfloat16), b.astype(jnp.bfloat16),
                               (dims, ((), ())), preferred_element_type=_F)


def _mm_kern(*refs, act, res):
    x_ref, w_ref, b_ref = refs[:3]
    o_ref = refs[-1]
    a = _dot(x_ref[...], w_ref[...]) + b_ref[...]
    if act == "gelu":
        a = 0.5 * a * (1.0 + jax.lax.erf(a * np.float32(1.0 / np.sqrt(2.0))))
    if res:
        a = refs[3][...] + a
    o_ref[...] = a


def _mm(x, w, b, act=None, res=None, bm=512, bn=512):
    M, Kd = x.shape
    N = w.shape[1]
    bm = min(bm, M)
    bn = min(bn, N)
    grid = (M // bm, N // bn)
    in_specs = [pl.BlockSpec((bm, Kd), lambda i, j: (i, 0)),
                pl.BlockSpec((Kd, bn), lambda i, j: (0, j)),
                pl.BlockSpec((1, bn), lambda i, j: (0, j))]
    args = [x, w, b.reshape(1, N)]
    if res is not None:
        in_specs.append(pl.BlockSpec((bm, bn), lambda i, j: (i, j)))
        args.append(res)
    return pl.pallas_call(
        functools.partial(_mm_kern, act=act, res=res is not None),
        grid=grid,
        in_specs=in_specs,
        out_specs=pl.BlockSpec((bm, bn), lambda i, j: (i, j)),
        out_shape=jax.ShapeDtypeStruct((M, N), _F),
    )(*args)


def _ln_kern(*refs, nin, res):
    o_ref = refs[-1]
    x = refs[0][...]
    for i in range(1, nin):
        x = x + refs[i][...]
    off = nin + (1 if res else 0)
    g = refs[off][...]
    bb = refs[off + 1][...]
    m = jnp.mean(x, axis=1, keepdims=True)
    v = jnp.mean((x - m) ** 2, axis=1, keepdims=True)
    y = (x - m) / jnp.sqrt(v + 1e-5) * g + bb
    if res:
        y = refs[nin][...] + y
    o_ref[...] = y


def _ln(xs, g, b, res=None, bm=512):
    M, Dd = xs[0].shape
    bm = min(bm, M)
    nin = len(xs)
    row = pl.BlockSpec((bm, Dd), lambda i: (i, 0))
    vec = pl.BlockSpec((1, Dd), lambda i: (0, 0))
    in_specs = [row] * nin
    args = list(xs)
    if res is not None:
        in_specs.append(row)
        args.append(res)
    in_specs += [vec, vec]
    args += [g.reshape(1, Dd), b.reshape(1, Dd)]
    return pl.pallas_call(
        functools.partial(_ln_kern, nin=nin, res=res is not None),
        grid=(M // bm,),
        in_specs=in_specs,
        out_specs=row,
        out_shape=jax.ShapeDtypeStruct((M, Dd), _F),
    )(*args)


def _attn_kern(q_ref, k_ref, v_ref, o_ref, *, kvalid):
    s = _dot(q_ref[0], k_ref[0], ((1,), (1,))) * np.float32(1.0 / np.sqrt(HD))
    if kvalid is not None:
        col = jax.lax.broadcasted_iota(jnp.int32, s.shape, 1)
        s = jnp.where(col < kvalid, s, -1e30)
    m = jnp.max(s, axis=1, keepdims=True)
    p = jnp.exp(s - m)
    o_ref[0] = _dot(p, v_ref[0]) / jnp.sum(p, axis=1, keepdims=True)


def _attn(q, k, v, kvalid=None, bq=512):
    BH, Tq, hd = q.shape
    Tk = k.shape[1]
    return pl.pallas_call(
        functools.partial(_attn_kern, kvalid=kvalid),
        grid=(BH, Tq // bq),
        in_specs=[pl.BlockSpec((1, bq, hd), lambda h, i: (h, i, 0)),
                  pl.BlockSpec((1, Tk, hd), lambda h, i: (h, 0, 0)),
                  pl.BlockSpec((1, Tk, hd), lambda h, i: (h, 0, 0))],
        out_specs=pl.BlockSpec((1, bq, hd), lambda h, i: (h, i, 0)),
        out_shape=jax.ShapeDtypeStruct((BH, Tq, hd), _F),
    )(q, k, v)


def _pool_kern(x_ref, w1t_ref, b1_ref, w2_ref, b2_ref, wot_ref, bo_ref, o_ref):
    xb = x_ref[0]
    t = jnp.tanh(_dot16(xb, w1t_ref[...], ((1,), (0,))) + b1_ref[...])
    sT = _dot16(w2_ref[...], t) + b2_ref[...]
    row = jax.lax.broadcasted_iota(jnp.int32, (NWP, T), 0)
    col = jax.lax.broadcasted_iota(jnp.int32, (NWP, T), 1)
    off = col - row * STRIDE
    valid = (off >= 0) & (off < WIN)
    S = jnp.where(valid, jnp.broadcast_to(sT, (NWP, T)), -1e30)
    m = jnp.max(S, axis=1, keepdims=True)
    p = jnp.exp(S - m)
    wts = p / jnp.sum(p, axis=1, keepdims=True)
    mean = _dot(wts, xb)
    ex2 = _dot(wts, xb * xb)
    std = jnp.sqrt(ex2 - mean * mean + 1e-6)
    cat = jnp.concatenate([mean, std], axis=1)
    o_ref[0] = _dot16(cat, wot_ref[...], ((1,), (0,))) + bo_ref[...]


def _pool(x, p):
    return pl.pallas_call(
        _pool_kern,
        grid=(B,),
        in_specs=[pl.BlockSpec((1, T, D), lambda b: (b, 0, 0)),
                  pl.BlockSpec((D, HID), lambda b: (0, 0)),
                  pl.BlockSpec((1, HID), lambda b: (0, 0)),
                  pl.BlockSpec((1, HID), lambda b: (0, 0)),
                  pl.BlockSpec((1, 1), lambda b: (0, 0)),
                  pl.BlockSpec((2 * D, D), lambda b: (0, 0)),
                  pl.BlockSpec((1, D), lambda b: (0, 0))],
        out_specs=pl.BlockSpec((1, NWP, D), lambda b: (b, 0, 0)),
        out_shape=jax.ShapeDtypeStruct((B, NWP, D), _F),
    )(x, p["W1"].T, p["b1"].reshape(1, HID), p["W2"].reshape(1, HID),
      p["b2"].reshape(1, 1), p["Wo"].T, p["bo"].reshape(1, D))


def _rvq_kern(r_ref, emb_ref, q_ref, i_ref, l_ref):
    r = r_ref[...]
    qout = jnp.zeros_like(r)
    loss = jnp.zeros((1, 1), _F)
    rowid = jax.lax.broadcasted_iota(jnp.int32, (BNWP, 1), 0)
    validrow = jax.lax.rem(rowid, NWP) < NW
    iota_k = jax.lax.broadcasted_iota(jnp.int32, (BNWP, K), 1)
    ones = jnp.ones((1, D), _F)
    idx_cols = []
    for l in range(L):
        e = emb_ref[l]
        ee = _dot(ones, e * e, ((1,), (1,)))
        rr = jnp.sum(r * r, axis=1, keepdims=True)
        re = _dot16(r, e)
        d = (rr + ee) - 2.0 * re
        dmin = jnp.min(d, axis=1, keepdims=True)
        ei = jnp.min(jnp.where(d == dmin, iota_k, K), axis=1, keepdims=True)
        oh = (iota_k == ei).astype(_F)
        q = _dot(oh, e)
        diff = q - r
        sq = jnp.where(validrow, jnp.sum(diff * diff, axis=1, keepdims=True), 0.0)
        loss = loss + 0.25 * (jnp.sum(sq, axis=0, keepdims=True)
                              / np.float32(B * NW * D))
        r = r - q
        qout = qout + q
        idx_cols.append(ei)
    q_ref[...] = qout
    i_ref[...] = jnp.concatenate(idx_cols, axis=1)
    l_ref[...] = loss


def _rvq(flat, emb):
    return pl.pallas_call(
        _rvq_kern,
        out_shape=(jax.ShapeDtypeStruct((BNWP, D), _F),
                   jax.ShapeDtypeStruct((BNWP, L), jnp.int32),
                   jax.ShapeDtypeStruct((1, 1), _F)),
    )(flat, emb)


def _heads(t, nb):
    return t.reshape(nb, T, NH, HD).transpose(0, 2, 1, 3).reshape(nb * NH, T, HD)


def _unheads(t, nb):
    return t.reshape(nb, NH, T, HD).transpose(0, 2, 1, 3).reshape(nb * T, D)


def _cross_kv(qflat, c):
    kv = _mm(qflat, c["Wqkv"][D:].T, c["bqkv"][D:], bm=BNWP)
    def side(a):
        a = a.reshape(B, NWP, NH, HD).transpose(0, 2, 1, 3)[:, :, :NW]
        a = jnp.pad(a, ((0, 0), (0, 0), (0, TKP - NW), (0, 0)))
        return a.reshape(B * NH, TKP, HD)
    return side(kv[:, :D]), side(kv[:, D:])


def _enc_block(xq, e, g0, b0, nb):
    xn = _ln([xq], g0, b0)
    qkv = _mm(xn, e["Wqkv"].T, e["bqkv"])
    q, k, v = (_heads(qkv[:, i * D:(i + 1) * D], nb) for i in range(3))
    sa = _mm(_unheads(_attn(q, k, v), nb), e["Wo"].T, e["bo"])
    h = _ln([xn, sa], e["g1"], e["be1"])
    ff = _mm(_mm(h, e["W1"].T, e["b1"], act="gelu"), e["W2"].T, e["b2"])
    return _ln([h, ff], e["g2"], e["be2"], res=xq)


def kernel(x, params):
    p = params
    codes_p = _pool(x, p["pool"])
    qout_f, idx_f, loss11 = _rvq(codes_p.reshape(BNWP, D), p["rvq_emb"])
    indices = idx_f.reshape(B, NWP, L)[:, :NW]
    loss = loss11.reshape(())

    blk0, blk1 = p["blocks"]

    u2 = _enc_block(p["pos"], blk0["enc"], blk0["n0g"], blk0["n0b"], 1)
    xn1 = _ln([u2], blk0["n1g"], blk0["n1b"])
    c = blk0["cross"]
    qc = _heads(_mm(xn1, c["Wqkv"][:D].T, c["bqkv"][:D]), 1)
    kc, vc = _cross_kv(qout_f, c)
    qb = jnp.broadcast_to(qc[None], (B, NH, T, HD)).reshape(B * NH, T, HD)
    co = _unheads(_attn(qb, kc, vc, kvalid=NW), B)
    u2b = jnp.broadcast_to(u2[None], (B, T, D)).reshape(B * T, D)
    xq = _mm(co, c["Wo"].T, c["bo"], res=u2b)
    xn = _ln([xq], blk0["n2g"], blk0["n2b"])
    f = blk0["ffn"]
    xq = _mm(_mm(xn, f["W1"].T, f["b1"], act="gelu"), f["W2"].T, f["b2"], res=xq)

    xq = _enc_block(xq, blk1["enc"], blk1["n0g"], blk1["n0b"], B)
    xn = _ln([xq], blk1["n1g"], blk1["n1b"])
    c = blk1["cross"]
    qc = _heads(_mm(xn, c["Wqkv"][:D].T, c["bqkv"][:D]), B)
    kc, vc = _cross_kv(qout_f, c)
    co = _unheads(_attn(qc, kc, vc, kvalid=NW), B)
    xq = _mm(co, c["Wo"].T, c["bo"], res=xq)
    xn = _ln([xq], blk1["n2g"], blk1["n2b"])
    f = blk1["ffn"]
    xq = _mm(_mm(xn, f["W1"].T, f["b1"], act="gelu"), f["W2"].T, f["b2"], res=xq)

    return xq.reshape(B, T, D), loss, indices


# speedup vs baseline: 2.2421x; 2.2421x over previous
import functools

import jax
import jax.numpy as jnp
import numpy as np
from jax.experimental import pallas as pl

B = 4; T = 2048; D = 512; NH = 8; HD = D // NH
K = 1024; L = 4; WIN = 25; STRIDE = 12; HID = 128
NW = (T - WIN) // STRIDE + 1
NWP = 176
TKP = 256
BNWP = B * NWP
_H = jax.lax.Precision.HIGHEST
_F = jnp.float32


def _dot(a, b, dims=None):
    if dims is None:
        return jnp.dot(a, b, preferred_element_type=_F, precision=_H)
    return jax.lax.dot_general(a, b, (dims, ((), ())),
                               preferred_element_type=_F, precision=_H)


def _dot16(a, b, dims=((1,), (1,))):
    return jax.lax.dot_general(a.astype(jnp.bfloat16), b.astype(jnp.bfloat16),
                               (dims, ((), ())), preferred_element_type=_F)


def _mm_kern(*refs, act, res):
    x_ref, w_ref, b_ref = refs[:3]
    o_ref = refs[-1]
    a = _dot16(x_ref[...], w_ref[...], ((1,), (0,))) + b_ref[...]
    if act == "gelu":
        a = 0.5 * a * (1.0 + jax.lax.erf(a * np.float32(1.0 / np.sqrt(2.0))))
    if res:
        a = refs[3][...] + a
    o_ref[...] = a


def _mm(x, w, b, act=None, res=None, bm=512, bn=512):
    M, Kd = x.shape
    N = w.shape[1]
    bm = min(bm, M)
    bn = min(bn, N)
    grid = (M // bm, N // bn)
    in_specs = [pl.BlockSpec((bm, Kd), lambda i, j: (i, 0)),
                pl.BlockSpec((Kd, bn), lambda i, j: (0, j)),
                pl.BlockSpec((1, bn), lambda i, j: (0, j))]
    args = [x, w, b.reshape(1, N)]
    if res is not None:
        in_specs.append(pl.BlockSpec((bm, bn), lambda i, j: (i, j)))
        args.append(res)
    return pl.pallas_call(
        functools.partial(_mm_kern, act=act, res=res is not None),
        grid=grid,
        in_specs=in_specs,
        out_specs=pl.BlockSpec((bm, bn), lambda i, j: (i, j)),
        out_shape=jax.ShapeDtypeStruct((M, N), _F),
    )(*args)


def _ln_kern(*refs, nin, res):
    o_ref = refs[-1]
    x = refs[0][...]
    for i in range(1, nin):
        x = x + refs[i][...]
    off = nin + (1 if res else 0)
    g = refs[off][...]
    bb = refs[off + 1][...]
    m = jnp.mean(x, axis=1, keepdims=True)
    v = jnp.mean((x - m) ** 2, axis=1, keepdims=True)
    y = (x - m) / jnp.sqrt(v + 1e-5) * g + bb
    if res:
        y = refs[nin][...] + y
    o_ref[...] = y


def _ln(xs, g, b, res=None, bm=512):
    M, Dd = xs[0].shape
    bm = min(bm, M)
    nin = len(xs)
    row = pl.BlockSpec((bm, Dd), lambda i: (i, 0))
    vec = pl.BlockSpec((1, Dd), lambda i: (0, 0))
    in_specs = [row] * nin
    args = list(xs)
    if res is not None:
        in_specs.append(row)
        args.append(res)
    in_specs += [vec, vec]
    args += [g.reshape(1, Dd), b.reshape(1, Dd)]
    return pl.pallas_call(
        functools.partial(_ln_kern, nin=nin, res=res is not None),
        grid=(M // bm,),
        in_specs=in_specs,
        out_specs=row,
        out_shape=jax.ShapeDtypeStruct((M, Dd), _F),
    )(*args)


def _attn_kern(q_ref, k_ref, v_ref, o_ref, *, kvalid):
    s = _dot16(q_ref[0], k_ref[0]) * np.float32(1.0 / np.sqrt(HD))
    if kvalid is not None:
        col = jax.lax.broadcasted_iota(jnp.int32, s.shape, 1)
        s = jnp.where(col < kvalid, s, -1e30)
    m = jnp.max(s, axis=1, keepdims=True)
    p = jnp.exp(s - m)
    o_ref[0] = _dot16(p, v_ref[0], ((1,), (0,))) / jnp.sum(p, axis=1, keepdims=True)


def _attn(q, k, v, kvalid=None, bq=512):
    BH, Tq, hd = q.shape
    Tk = k.shape[1]
    return pl.pallas_call(
        functools.partial(_attn_kern, kvalid=kvalid),
        grid=(BH, Tq // bq),
        in_specs=[pl.BlockSpec((1, bq, hd), lambda h, i: (h, i, 0)),
                  pl.BlockSpec((1, Tk, hd), lambda h, i: (h, 0, 0)),
                  pl.BlockSpec((1, Tk, hd), lambda h, i: (h, 0, 0))],
        out_specs=pl.BlockSpec((1, bq, hd), lambda h, i: (h, i, 0)),
        out_shape=jax.ShapeDtypeStruct((BH, Tq, hd), _F),
    )(q, k, v)


def _pool_kern(x_ref, w1t_ref, b1_ref, w2_ref, b2_ref, wot_ref, bo_ref, o_ref):
    xb = x_ref[0]
    t = jnp.tanh(_dot16(xb, w1t_ref[...], ((1,), (0,))) + b1_ref[...])
    sT = _dot16(w2_ref[...], t) + b2_ref[...]
    row = jax.lax.broadcasted_iota(jnp.int32, (NWP, T), 0)
    col = jax.lax.broadcasted_iota(jnp.int32, (NWP, T), 1)
    off = col - row * STRIDE
    valid = (off >= 0) & (off < WIN)
    S = jnp.where(valid, jnp.broadcast_to(sT, (NWP, T)), -1e30)
    m = jnp.max(S, axis=1, keepdims=True)
    p = jnp.exp(S - m)
    wts = p / jnp.sum(p, axis=1, keepdims=True)
    mean = _dot(wts, xb)
    ex2 = _dot(wts, xb * xb)
    std = jnp.sqrt(ex2 - mean * mean + 1e-6)
    cat = jnp.concatenate([mean, std], axis=1)
    o_ref[0] = _dot16(cat, wot_ref[...], ((1,), (0,))) + bo_ref[...]


def _pool(x, p):
    return pl.pallas_call(
        _pool_kern,
        grid=(B,),
        in_specs=[pl.BlockSpec((1, T, D), lambda b: (b, 0, 0)),
                  pl.BlockSpec((D, HID), lambda b: (0, 0)),
                  pl.BlockSpec((1, HID), lambda b: (0, 0)),
                  pl.BlockSpec((1, HID), lambda b: (0, 0)),
                  pl.BlockSpec((1, 1), lambda b: (0, 0)),
                  pl.BlockSpec((2 * D, D), lambda b: (0, 0)),
                  pl.BlockSpec((1, D), lambda b: (0, 0))],
        out_specs=pl.BlockSpec((1, NWP, D), lambda b: (b, 0, 0)),
        out_shape=jax.ShapeDtypeStruct((B, NWP, D), _F),
    )(x, p["W1"].T, p["b1"].reshape(1, HID), p["W2"].reshape(1, HID),
      p["b2"].reshape(1, 1), p["Wo"].T, p["bo"].reshape(1, D))


def _rvq_kern(r_ref, emb_ref, q_ref, i_ref, l_ref):
    r = r_ref[...]
    qout = jnp.zeros_like(r)
    loss = jnp.zeros((1, 1), _F)
    rowid = jax.lax.broadcasted_iota(jnp.int32, (BNWP, 1), 0)
    validrow = jax.lax.rem(rowid, NWP) < NW
    iota_k = jax.lax.broadcasted_iota(jnp.int32, (BNWP, K), 1)
    ones = jnp.ones((1, D), _F)
    idx_cols = []
    for l in range(L):
        e = emb_ref[l]
        ee = _dot(ones, e * e, ((1,), (1,)))
        rr = jnp.sum(r * r, axis=1, keepdims=True)
        re = _dot16(r, e)
        d = (rr + ee) - 2.0 * re
        dmin = jnp.min(d, axis=1, keepdims=True)
        ei = jnp.min(jnp.where(d == dmin, iota_k, K), axis=1, keepdims=True)
        oh = (iota_k == ei).astype(_F)
        q = _dot(oh, e)
        diff = q - r
        sq = jnp.where(validrow, jnp.sum(diff * diff, axis=1, keepdims=True), 0.0)
        loss = loss + 0.25 * (jnp.sum(sq, axis=0, keepdims=True)
                              / np.float32(B * NW * D))
        r = r - q
        qout = qout + q
        idx_cols.append(ei)
    q_ref[...] = qout
    i_ref[...] = jnp.concatenate(idx_cols, axis=1)
    l_ref[...] = loss


def _rvq(flat, emb):
    return pl.pallas_call(
        _rvq_kern,
        out_shape=(jax.ShapeDtypeStruct((BNWP, D), _F),
                   jax.ShapeDtypeStruct((BNWP, L), jnp.int32),
                   jax.ShapeDtypeStruct((1, 1), _F)),
    )(flat, emb)


def _heads(t, nb):
    return t.reshape(nb, T, NH, HD).transpose(0, 2, 1, 3).reshape(nb * NH, T, HD)


def _unheads(t, nb):
    return t.reshape(nb, NH, T, HD).transpose(0, 2, 1, 3).reshape(nb * T, D)


def _cross_kv(qflat, c):
    kv = _mm(qflat, c["Wqkv"][D:].T, c["bqkv"][D:], bm=BNWP)
    def side(a):
        a = a.reshape(B, NWP, NH, HD).transpose(0, 2, 1, 3)[:, :, :NW]
        a = jnp.pad(a, ((0, 0), (0, 0), (0, TKP - NW), (0, 0)))
        return a.reshape(B * NH, TKP, HD)
    return side(kv[:, :D]), side(kv[:, D:])


def _enc_block(xq, e, g0, b0, nb):
    xn = _ln([xq], g0, b0)
    qkv = _mm(xn, e["Wqkv"].T, e["bqkv"])
    q, k, v = (_heads(qkv[:, i * D:(i + 1) * D], nb) for i in range(3))
    sa = _mm(_unheads(_attn(q, k, v), nb), e["Wo"].T, e["bo"])
    h = _ln([xn, sa], e["g1"], e["be1"])
    ff = _mm(_mm(h, e["W1"].T, e["b1"], act="gelu"), e["W2"].T, e["b2"])
    return _ln([h, ff], e["g2"], e["be2"], res=xq)


def kernel(x, params):
    p = params
    codes_p = _pool(x, p["pool"])
    qout_f, idx_f, loss11 = _rvq(codes_p.reshape(BNWP, D), p["rvq_emb"])
    indices = idx_f.reshape(B, NWP, L)[:, :NW]
    loss = loss11.reshape(())

    blk0, blk1 = p["blocks"]

    u2 = _enc_block(p["pos"], blk0["enc"], blk0["n0g"], blk0["n0b"], 1)
    xn1 = _ln([u2], blk0["n1g"], blk0["n1b"])
    c = blk0["cross"]
    qc = _heads(_mm(xn1, c["Wqkv"][:D].T, c["bqkv"][:D]), 1)
    kc, vc = _cross_kv(qout_f, c)
    qb = jnp.broadcast_to(qc[None], (B, NH, T, HD)).reshape(B * NH, T, HD)
    co = _unheads(_attn(qb, kc, vc, kvalid=NW), B)
    u2b = jnp.broadcast_to(u2[None], (B, T, D)).reshape(B * T, D)
    xq = _mm(co, c["Wo"].T, c["bo"], res=u2b)
    xn = _ln([xq], blk0["n2g"], blk0["n2b"])
    f = blk0["ffn"]
    xq = _mm(_mm(xn, f["W1"].T, f["b1"], act="gelu"), f["W2"].T, f["b2"], res=xq)

    xq = _enc_block(xq, blk1["enc"], blk1["n0g"], blk1["n0b"], B)
    xn = _ln([xq], blk1["n1g"], blk1["n1b"])
    c = blk1["cross"]
    qc = _heads(_mm(xn, c["Wqkv"][:D].T, c["bqkv"][:D]), B)
    kc, vc = _cross_kv(qout_f, c)
    co = _unheads(_attn(qc, kc, vc, kvalid=NW), B)
    xq = _mm(co, c["Wo"].T, c["bo"], res=xq)
    xn = _ln([xq], blk1["n2g"], blk1["n2b"])
    f = blk1["ffn"]
    xq = _mm(_mm(xn, f["W1"].T, f["b1"], act="gelu"), f["W2"].T, f["b2"], res=xq)

    return xq.reshape(B, T, D), loss, indices


# speedup vs baseline: 2.7094x; 1.2084x over previous
import functools

import jax
import jax.numpy as jnp
import numpy as np
from jax.experimental import pallas as pl

B = 4; T = 2048; D = 512; NH = 8; HD = D // NH
K = 1024; L = 4; WIN = 25; STRIDE = 12; HID = 128
NW = (T - WIN) // STRIDE + 1
NWP = 176
BNWP = B * NWP
_H = jax.lax.Precision.HIGHEST
_F = jnp.float32


def _dot(a, b, dims=None):
    if dims is None:
        return jnp.dot(a, b, preferred_element_type=_F, precision=_H)
    return jax.lax.dot_general(a, b, (dims, ((), ())),
                               preferred_element_type=_F, precision=_H)


def _dot16(a, b, dims=((1,), (1,))):
    return jax.lax.dot_general(a.astype(jnp.bfloat16), b.astype(jnp.bfloat16),
                               (dims, ((), ())), preferred_element_type=_F)


def _lnf(x, g, b):
    m = jnp.mean(x, axis=1, keepdims=True)
    v = jnp.mean((x - m) ** 2, axis=1, keepdims=True)
    return (x - m) / jnp.sqrt(v + 1e-5) * g + b


def _gelu(a):
    return 0.5 * a * (1.0 + jax.lax.erf(a * np.float32(1.0 / np.sqrt(2.0))))


def _mm_kern(*refs, act, ln, res):
    x_ref, w_ref, b_ref = refs[:3]
    i = 3
    if ln:
        g_ref, be_ref = refs[i:i + 2]
        i += 2
    o_ref = refs[-1]
    x = x_ref[...]
    if ln:
        x = _lnf(x, g_ref[...], be_ref[...])
    a = _dot16(x, w_ref[...], ((1,), (0,))) + b_ref[...]
    if act == "gelu":
        a = _gelu(a)
    if res:
        a = refs[i][...] + a
    o_ref[...] = a


def _mm(x, w, b, act=None, ln=None, res=None, bm=512):
    M, Kd = x.shape
    N = w.shape[1]
    bm = min(bm, M)
    row = pl.BlockSpec((bm, Kd), lambda i: (i, 0))
    in_specs = [row,
                pl.BlockSpec((Kd, N), lambda i: (0, 0)),
                pl.BlockSpec((1, N), lambda i: (0, 0))]
    args = [x, w, b.reshape(1, N)]
    if ln is not None:
        vec = pl.BlockSpec((1, Kd), lambda i: (0, 0))
        in_specs += [vec, vec]
        args += [ln[0].reshape(1, Kd), ln[1].reshape(1, Kd)]
    if res is not None:
        rrows = res.shape[0]
        nrep = rrows // bm
        in_specs.append(pl.BlockSpec((bm, N), lambda i, n=nrep: (i % n, 0)))
        args.append(res)
    return pl.pallas_call(
        functools.partial(_mm_kern, act=act, ln=ln is not None,
                          res=res is not None),
        grid=(M // bm,),
        in_specs=in_specs,
        out_specs=pl.BlockSpec((bm, N), lambda i: (i, 0)),
        out_shape=jax.ShapeDtypeStruct((M, N), _F),
    )(*args)


def _oproj_kern(ao_ref, w_ref, b_ref, xq_ref, g0_ref, b0_ref, g1_ref, b1_ref,
                o_ref):
    a = _dot16(ao_ref[...], w_ref[...], ((1,), (0,))) + b_ref[...]
    xn = _lnf(xq_ref[...], g0_ref[...], b0_ref[...])
    o_ref[...] = _lnf(xn + a, g1_ref[...], b1_ref[...])


def _oproj(ao, w, b, xq, g0, b0, g1, b1, bm=512):
    M = ao.shape[0]
    row = pl.BlockSpec((bm, D), lambda i: (i, 0))
    vec = pl.BlockSpec((1, D), lambda i: (0, 0))
    return pl.pallas_call(
        _oproj_kern,
        grid=(M // bm,),
        in_specs=[row, pl.BlockSpec((D, D), lambda i: (0, 0)), vec, row,
                  vec, vec, vec, vec],
        out_specs=row,
        out_shape=jax.ShapeDtypeStruct((M, D), _F),
    )(ao, w, b.reshape(1, D), xq, g0.reshape(1, D), b0.reshape(1, D),
      g1.reshape(1, D), b1.reshape(1, D))


def _ffn2_kern(x_ref, w_ref, b_ref, h_ref, xq_ref, g_ref, be_ref, o_ref):
    a = _dot16(x_ref[...], w_ref[...], ((1,), (0,))) + b_ref[...]
    o_ref[...] = xq_ref[...] + _lnf(h_ref[...] + a, g_ref[...], be_ref[...])


def _ffn2(x, w, b, h, xq, g, be, bm=512):
    M, Kd = x.shape
    rowk = pl.BlockSpec((bm, Kd), lambda i: (i, 0))
    row = pl.BlockSpec((bm, D), lambda i: (i, 0))
    vec = pl.BlockSpec((1, D), lambda i: (0, 0))
    return pl.pallas_call(
        _ffn2_kern,
        grid=(M // bm,),
        in_specs=[rowk, pl.BlockSpec((Kd, D), lambda i: (0, 0)), vec,
                  row, row, vec, vec],
        out_specs=row,
        out_shape=jax.ShapeDtypeStruct((M, D), _F),
    )(x, w, b.reshape(1, D), h, xq, g.reshape(1, D), be.reshape(1, D))


def _attn_kern(q_ref, k_ref, v_ref, o_ref, *, kvalid):
    s = _dot16(q_ref[0], k_ref[0]) * np.float32(1.0 / np.sqrt(HD))
    if kvalid is not None:
        col = jax.lax.broadcasted_iota(jnp.int32, s.shape, 1)
        s = jnp.where(col < kvalid, s, -1e30)
    m = jnp.max(s, axis=1, keepdims=True)
    p = jnp.exp(s - m)
    o_ref[0] = _dot16(p, v_ref[0], ((1,), (0,))) / jnp.sum(p, axis=1, keepdims=True)


def _attn(q, k, v, kvalid=None, bq=512):
    nq = q.shape[0]
    BH = k.shape[0]
    Tq = q.shape[1]
    Tk = k.shape[1]
    hd = q.shape[2]
    qmap = (lambda h, i: (h, i, 0)) if nq == BH else \
           (lambda h, i: (jax.lax.rem(h, nq), i, 0))
    return pl.pallas_call(
        functools.partial(_attn_kern, kvalid=kvalid),
        grid=(BH, Tq // bq),
        in_specs=[pl.BlockSpec((1, bq, hd), qmap),
                  pl.BlockSpec((1, Tk, hd), lambda h, i: (h, 0, 0)),
                  pl.BlockSpec((1, Tk, hd), lambda h, i: (h, 0, 0))],
        out_specs=pl.BlockSpec((1, bq, hd), lambda h, i: (h, i, 0)),
        out_shape=jax.ShapeDtypeStruct((BH, Tq, hd), _F),
    )(q, k, v)


def _pool_kern(x_ref, w1t_ref, b1_ref, w2_ref, b2_ref, wot_ref, bo_ref, o_ref):
    xb = x_ref[0]
    t = jnp.tanh(_dot16(xb, w1t_ref[...], ((1,), (0,))) + b1_ref[...])
    sT = _dot16(w2_ref[...], t) + b2_ref[...]
    row = jax.lax.broadcasted_iota(jnp.int32, (NWP, T), 0)
    col = jax.lax.broadcasted_iota(jnp.int32, (NWP, T), 1)
    off = col - row * STRIDE
    valid = (off >= 0) & (off < WIN)
    S = jnp.where(valid, jnp.broadcast_to(sT, (NWP, T)), -1e30)
    m = jnp.max(S, axis=1, keepdims=True)
    p = jnp.exp(S - m)
    wts = p / jnp.sum(p, axis=1, keepdims=True)
    mean = _dot(wts, xb)
    ex2 = _dot(wts, xb * xb)
    std = jnp.sqrt(ex2 - mean * mean + 1e-6)
    cat = jnp.concatenate([mean, std], axis=1)
    o_ref[0] = _dot16(cat, wot_ref[...], ((1,), (0,))) + bo_ref[...]


def _pool(x, p):
    return pl.pallas_call(
        _pool_kern,
        grid=(B,),
        in_specs=[pl.BlockSpec((1, T, D), lambda b: (b, 0, 0)),
                  pl.BlockSpec((D, HID), lambda b: (0, 0)),
                  pl.BlockSpec((1, HID), lambda b: (0, 0)),
                  pl.BlockSpec((1, HID), lambda b: (0, 0)),
                  pl.BlockSpec((1, 1), lambda b: (0, 0)),
                  pl.BlockSpec((2 * D, D), lambda b: (0, 0)),
                  pl.BlockSpec((1, D), lambda b: (0, 0))],
        out_specs=pl.BlockSpec((1, NWP, D), lambda b: (b, 0, 0)),
        out_shape=jax.ShapeDtypeStruct((B, NWP, D), _F),
    )(x, p["W1"].T, p["b1"].reshape(1, HID), p["W2"].reshape(1, HID),
      p["b2"].reshape(1, 1), p["Wo"].T, p["bo"].reshape(1, D))


def _rvq_kern(r_ref, emb_ref, q_ref, i_ref, l_ref):
    r = r_ref[...]
    qout = jnp.zeros_like(r)
    loss = jnp.zeros((1, 1), _F)
    rowid = jax.lax.broadcasted_iota(jnp.int32, (BNWP, 1), 0)
    validrow = jax.lax.rem(rowid, NWP) < NW
    iota_k = jax.lax.broadcasted_iota(jnp.int32, (BNWP, K), 1)
    ones = jnp.ones((1, D), _F)
    idx_cols = []
    for l in range(L):
        e = emb_ref[l]
        ee = _dot(ones, e * e, ((1,), (1,)))
        rr = jnp.sum(r * r, axis=1, keepdims=True)
        re = _dot16(r, e)
        d = (rr + ee) - 2.0 * re
        dmin = jnp.min(d, axis=1, keepdims=True)
        ei = jnp.min(jnp.where(d == dmin, iota_k, K), axis=1, keepdims=True)
        oh = (iota_k == ei).astype(_F)
        q = _dot(oh, e)
        diff = q - r
        sq = jnp.where(validrow, jnp.sum(diff * diff, axis=1, keepdims=True), 0.0)
        loss = loss + 0.25 * (jnp.sum(sq, axis=0, keepdims=True)
                              / np.float32(B * NW * D))
        r = r - q
        qout = qout + q
        idx_cols.append(ei)
    q_ref[...] = qout
    i_ref[...] = jnp.concatenate(idx_cols, axis=1)
    l_ref[...] = loss


def _rvq(flat, emb):
    return pl.pallas_call(
        _rvq_kern,
        out_shape=(jax.ShapeDtypeStruct((BNWP, D), _F),
                   jax.ShapeDtypeStruct((BNWP, L), jnp.int32),
                   jax.ShapeDtypeStruct((1, 1), _F)),
    )(flat, emb)


def _heads(t, nb):
    return t.reshape(nb, T, NH, HD).transpose(0, 2, 1, 3).reshape(nb * NH, T, HD)


def _unheads(t, nb):
    return t.reshape(nb, NH, T, HD).transpose(0, 2, 1, 3).reshape(nb * T, D)


def _cross_kv(qflat, c):
    kv = _mm(qflat, c["Wqkv"][D:].T, c["bqkv"][D:], bm=BNWP)
    def side(a):
        return a.reshape(B, NWP, NH, HD).transpose(0, 2, 1, 3).reshape(
            B * NH, NWP, HD)
    return side(kv[:, :D]), side(kv[:, D:])


def _enc_block(xq, e, g0, b0, nb):
    qkv = _mm(xq, e["Wqkv"].T, e["bqkv"], ln=(g0, b0))
    q, k, v = (_heads(qkv[:, i * D:(i + 1) * D], nb) for i in range(3))
    h = _oproj(_unheads(_attn(q, k, v), nb), e["Wo"].T, e["bo"],
               xq, g0, b0, e["g1"], e["be1"])
    g = _mm(h, e["W1"].T, e["b1"], act="gelu")
    return _ffn2(g, e["W2"].T, e["b2"], h, xq, e["g2"], e["be2"])


def _cross_ffn(xq, blk, qout_f, nb):
    c = blk["cross"]
    qc = _mm(xq, c["Wqkv"][:D].T, c["bqkv"][:D], ln=(blk["n1g"], blk["n1b"]))
    kc, vc = _cross_kv(qout_f, c)
    co = _unheads(_attn(_heads(qc, nb), kc, vc, kvalid=NW), B)
    xq = _mm(co, c["Wo"].T, c["bo"], res=xq)
    f = blk["ffn"]
    g = _mm(xq, f["W1"].T, f["b1"], act="gelu", ln=(blk["n2g"], blk["n2b"]))
    return _mm(g, f["W2"].T, f["b2"], res=xq)


def kernel(x, params):
    p = params
    codes_p = _pool(x, p["pool"])
    qout_f, idx_f, loss11 = _rvq(codes_p.reshape(BNWP, D), p["rvq_emb"])
    indices = idx_f.reshape(B, NWP, L)[:, :NW]
    loss = loss11.reshape(())

    blk0, blk1 = p["blocks"]

    u2 = _enc_block(p["pos"], blk0["enc"], blk0["n0g"], blk0["n0b"], 1)
    c = blk0["cross"]
    qc = _mm(u2, c["Wqkv"][:D].T, c["bqkv"][:D], ln=(blk0["n1g"], blk0["n1b"]))
    kc, vc = _cross_kv(qout_f, c)
    co = _unheads(_attn(_heads(qc, 1), kc, vc, kvalid=NW), B)
    xq = _mm(co, c["Wo"].T, c["bo"], res=u2)
    f = blk0["ffn"]
    g = _mm(xq, f["W1"].T, f["b1"], act="gelu", ln=(blk0["n2g"], blk0["n2b"]))
    xq = _mm(g, f["W2"].T, f["b2"], res=xq)

    xq = _enc_block(xq, blk1["enc"], blk1["n0g"], blk1["n0b"], B)
    xq = _cross_ffn(xq, blk1, qout_f, B)

    return xq.reshape(B, T, D), loss, indices


# speedup vs baseline: 2.9079x; 1.0733x over previous
import functools

import jax
import jax.numpy as jnp
import numpy as np
from jax.experimental import pallas as pl

B = 4; T = 2048; D = 512; NH = 8; HD = D // NH
K = 1024; L = 4; WIN = 25; STRIDE = 12; HID = 128
NW = (T - WIN) // STRIDE + 1
NWP = 176
BNWP = B * NWP
_H = jax.lax.Precision.HIGHEST
_F = jnp.float32
_SCALE = np.float32(1.0 / np.sqrt(HD))


def _dot(a, b, dims=None):
    if dims is None:
        return jnp.dot(a, b, preferred_element_type=_F, precision=_H)
    return jax.lax.dot_general(a, b, (dims, ((), ())),
                               preferred_element_type=_F, precision=_H)


def _dot16(a, b, dims=((1,), (1,))):
    return jax.lax.dot_general(a.astype(jnp.bfloat16), b.astype(jnp.bfloat16),
                               (dims, ((), ())), preferred_element_type=_F)


def _lnf(x, g, b):
    m = jnp.mean(x, axis=1, keepdims=True)
    v = jnp.mean((x - m) ** 2, axis=1, keepdims=True)
    return (x - m) / jnp.sqrt(v + 1e-5) * g + b


def _gelu(a):
    return 0.5 * a * (1.0 + jax.lax.erf(a * np.float32(1.0 / np.sqrt(2.0))))


def _mha_heads(q2, k2, v2, mask_cols=None):
    outs = []
    for h in range(NH):
        sl = slice(h * HD, (h + 1) * HD)
        s = _dot16(q2[:, sl], k2[:, sl]) * _SCALE
        if mask_cols is not None:
            col = jax.lax.broadcasted_iota(jnp.int32, s.shape, 1)
            s = jnp.where(col < mask_cols, s, -1e30)
        m = jnp.max(s, axis=1, keepdims=True)
        p = jnp.exp(s - m)
        outs.append(_dot16(p, v2[:, sl], ((1,), (0,)))
                    / jnp.sum(p, axis=1, keepdims=True))
    return jnp.concatenate(outs, axis=1)


def _mm_kern(*refs, act, ln, res):
    x_ref, w_ref, b_ref = refs[:3]
    i = 3
    if ln:
        g_ref, be_ref = refs[i:i + 2]
        i += 2
    o_ref = refs[-1]
    x = x_ref[...]
    if ln:
        x = _lnf(x, g_ref[...], be_ref[...])
    a = _dot16(x, w_ref[...], ((1,), (0,))) + b_ref[...]
    if act == "gelu":
        a = _gelu(a)
    if res:
        a = refs[i][...] + a
    o_ref[...] = a


def _mm(x, w, b, act=None, ln=None, res=None, bm=512):
    M, Kd = x.shape
    N = w.shape[1]
    bm = min(bm, M)
    row = pl.BlockSpec((bm, Kd), lambda i: (i, 0))
    in_specs = [row,
                pl.BlockSpec((Kd, N), lambda i: (0, 0)),
                pl.BlockSpec((1, N), lambda i: (0, 0))]
    args = [x, w, b.reshape(1, N)]
    if ln is not None:
        vec = pl.BlockSpec((1, Kd), lambda i: (0, 0))
        in_specs += [vec, vec]
        args += [ln[0].reshape(1, Kd), ln[1].reshape(1, Kd)]
    if res is not None:
        nrep = res.shape[0] // bm
        in_specs.append(pl.BlockSpec((bm, N), lambda i, n=nrep: (i % n, 0)))
        args.append(res)
    return pl.pallas_call(
        functools.partial(_mm_kern, act=act, ln=ln is not None,
                          res=res is not None),
        grid=(M // bm,),
        in_specs=in_specs,
        out_specs=pl.BlockSpec((bm, N), lambda i: (i, 0)),
        out_shape=jax.ShapeDtypeStruct((M, N), _F),
    )(*args)


def _ffn2_kern(x_ref, w_ref, b_ref, h_ref, xq_ref, g_ref, be_ref, o_ref):
    a = _dot16(x_ref[...], w_ref[...], ((1,), (0,))) + b_ref[...]
    o_ref[...] = xq_ref[...] + _lnf(h_ref[...] + a, g_ref[...], be_ref[...])


def _ffn2(x, w, b, h, xq, g, be, bm=512):
    M, Kd = x.shape
    rowk = pl.BlockSpec((bm, Kd), lambda i: (i, 0))
    row = pl.BlockSpec((bm, D), lambda i: (i, 0))
    vec = pl.BlockSpec((1, D), lambda i: (0, 0))
    return pl.pallas_call(
        _ffn2_kern,
        grid=(M // bm,),
        in_specs=[rowk, pl.BlockSpec((Kd, D), lambda i: (0, 0)), vec,
                  row, row, vec, vec],
        out_specs=row,
        out_shape=jax.ShapeDtypeStruct((M, D), _F),
    )(x, w, b.reshape(1, D), h, xq, g.reshape(1, D), be.reshape(1, D))


def _sattn_kern(q_ref, k_ref, v_ref, xq_ref, wo_ref, bo_ref,
                g0_ref, b0_ref, g1_ref, b1_ref, o_ref):
    ao = _mha_heads(q_ref[0, :, 0, 0, :], k_ref[0, :, 1, 0, :],
                    v_ref[0, :, 2, 0, :])
    a = _dot16(ao, wo_ref[...], ((1,), (0,))) + bo_ref[...]
    xn = _lnf(xq_ref[...], g0_ref[...], b0_ref[...])
    o_ref[...] = _lnf(xn + a, g1_ref[...], b1_ref[...])


def _sattn(qkv, xq, wo, bo, g0, b0, g1, b1, nb, bq=512):
    qkv5 = qkv.reshape(nb, T, 3, 1, D)
    nqt = T // bq
    row = pl.BlockSpec((bq, D), lambda b, i, n=nqt: (b * n + i, 0))
    vec = pl.BlockSpec((1, D), lambda b, i: (0, 0))
    return pl.pallas_call(
        _sattn_kern,
        grid=(nb, nqt),
        in_specs=[pl.BlockSpec((1, bq, 1, 1, D), lambda b, i: (b, i, 0, 0, 0)),
                  pl.BlockSpec((1, T, 1, 1, D), lambda b, i: (b, 0, 1, 0, 0)),
                  pl.BlockSpec((1, T, 1, 1, D), lambda b, i: (b, 0, 2, 0, 0)),
                  row, pl.BlockSpec((D, D), lambda b, i: (0, 0)), vec,
                  vec, vec, vec, vec],
        out_specs=row,
        out_shape=jax.ShapeDtypeStruct((nb * T, D), _F),
    )(qkv5, qkv5, qkv5, xq, wo, bo.reshape(1, D), g0.reshape(1, D),
      b0.reshape(1, D), g1.reshape(1, D), b1.reshape(1, D))


def _cattn_kern(q_ref, k_ref, v_ref, xq_ref, wo_ref, bo_ref, o_ref):
    ao = _mha_heads(q_ref[...], k_ref[0, :, 0, 0, :], v_ref[0, :, 1, 0, :],
                    mask_cols=NW)
    o_ref[...] = xq_ref[...] + _dot16(ao, wo_ref[...], ((1,), (0,))) + bo_ref[...]


def _cattn(qc, kv, wo, bo, xq, bq=512):
    kv5 = kv.reshape(B, NWP, 2, 1, D)
    nqt = T // bq
    nq = qc.shape[0] // bq
    nr = xq.shape[0] // bq
    qrow = pl.BlockSpec((bq, D), lambda b, i, n=nqt, m=nq: ((b * n + i) % m, 0))
    rrow = pl.BlockSpec((bq, D), lambda b, i, n=nqt, m=nr: ((b * n + i) % m, 0))
    orow = pl.BlockSpec((bq, D), lambda b, i, n=nqt: (b * n + i, 0))
    vec = pl.BlockSpec((1, D), lambda b, i: (0, 0))
    return pl.pallas_call(
        _cattn_kern,
        grid=(B, nqt),
        in_specs=[qrow,
                  pl.BlockSpec((1, NWP, 1, 1, D), lambda b, i: (b, 0, 0, 0, 0)),
                  pl.BlockSpec((1, NWP, 1, 1, D), lambda b, i: (b, 0, 1, 0, 0)),
                  rrow, pl.BlockSpec((D, D), lambda b, i: (0, 0)), vec],
        out_specs=orow,
        out_shape=jax.ShapeDtypeStruct((B * T, D), _F),
    )(qc, kv5, kv5, xq, wo, bo.reshape(1, D))


def _pool_kern(x_ref, w1t_ref, b1_ref, w2_ref, b2_ref, wot_ref, bo_ref, o_ref):
    xb = x_ref[0]
    t = jnp.tanh(_dot16(xb, w1t_ref[...], ((1,), (0,))) + b1_ref[...])
    sT = _dot16(w2_ref[...], t) + b2_ref[...]
    row = jax.lax.broadcasted_iota(jnp.int32, (NWP, T), 0)
    col = jax.lax.broadcasted_iota(jnp.int32, (NWP, T), 1)
    off = col - row * STRIDE
    valid = (off >= 0) & (off < WIN)
    S = jnp.where(valid, jnp.broadcast_to(sT, (NWP, T)), -1e30)
    m = jnp.max(S, axis=1, keepdims=True)
    p = jnp.exp(S - m)
    wts = p / jnp.sum(p, axis=1, keepdims=True)
    mean = _dot(wts, xb)
    ex2 = _dot(wts, xb * xb)
    std = jnp.sqrt(ex2 - mean * mean + 1e-6)
    cat = jnp.concatenate([mean, std], axis=1)
    o_ref[0] = _dot16(cat, wot_ref[...], ((1,), (0,))) + bo_ref[...]


def _pool(x, p):
    return pl.pallas_call(
        _pool_kern,
        grid=(B,),
        in_specs=[pl.BlockSpec((1, T, D), lambda b: (b, 0, 0)),
                  pl.BlockSpec((D, HID), lambda b: (0, 0)),
                  pl.BlockSpec((1, HID), lambda b: (0, 0)),
                  pl.BlockSpec((1, HID), lambda b: (0, 0)),
                  pl.BlockSpec((1, 1), lambda b: (0, 0)),
                  pl.BlockSpec((2 * D, D), lambda b: (0, 0)),
                  pl.BlockSpec((1, D), lambda b: (0, 0))],
        out_specs=pl.BlockSpec((1, NWP, D), lambda b: (b, 0, 0)),
        out_shape=jax.ShapeDtypeStruct((B, NWP, D), _F),
    )(x, p["W1"].T, p["b1"].reshape(1, HID), p["W2"].reshape(1, HID),
      p["b2"].reshape(1, 1), p["Wo"].T, p["bo"].reshape(1, D))


def _rvq_kern(r_ref, emb_ref, q_ref, i_ref, l_ref):
    r = r_ref[...]
    qout = jnp.zeros_like(r)
    loss = jnp.zeros((1, 1), _F)
    rowid = jax.lax.broadcasted_iota(jnp.int32, (BNWP, 1), 0)
    validrow = jax.lax.rem(rowid, NWP) < NW
    iota_k = jax.lax.broadcasted_iota(jnp.int32, (BNWP, K), 1)
    ones = jnp.ones((1, D), _F)
    idx_cols = []
    for l in range(L):
        e = emb_ref[l]
        ee = _dot(ones, e * e, ((1,), (1,)))
        rr = jnp.sum(r * r, axis=1, keepdims=True)
        re = _dot16(r, e)
        d = (rr + ee) - 2.0 * re
        dmin = jnp.min(d, axis=1, keepdims=True)
        ei = jnp.min(jnp.where(d == dmin, iota_k, K), axis=1, keepdims=True)
        oh = (iota_k == ei).astype(_F)
        q = _dot(oh, e)
        diff = q - r
        sq = jnp.where(validrow, jnp.sum(diff * diff, axis=1, keepdims=True), 0.0)
        loss = loss + 0.25 * (jnp.sum(sq, axis=0, keepdims=True)
                              / np.float32(B * NW * D))
        r = r - q
        qout = qout + q
        idx_cols.append(ei)
    q_ref[...] = qout
    i_ref[...] = jnp.concatenate(idx_cols, axis=1)
    l_ref[...] = loss


def _rvq(flat, emb):
    return pl.pallas_call(
        _rvq_kern,
        out_shape=(jax.ShapeDtypeStruct((BNWP, D), _F),
                   jax.ShapeDtypeStruct((BNWP, L), jnp.int32),
                   jax.ShapeDtypeStruct((1, 1), _F)),
    )(flat, emb)


def _enc_block(xq, e, g0, b0, nb):
    qkv = _mm(xq, e["Wqkv"].T, e["bqkv"], ln=(g0, b0))
    h = _sattn(qkv, xq, e["Wo"].T, e["bo"], g0, b0, e["g1"], e["be1"], nb)
    g = _mm(h, e["W1"].T, e["b1"], act="gelu")
    return _ffn2(g, e["W2"].T, e["b2"], h, xq, e["g2"], e["be2"])


def _cross_ffn(xq, blk, qout_f):
    c = blk["cross"]
    qc = _mm(xq, c["Wqkv"][:D].T, c["bqkv"][:D], ln=(blk["n1g"], blk["n1b"]))
    kv = _mm(qout_f, c["Wqkv"][D:].T, c["bqkv"][D:], bm=BNWP)
    xq2 = _cattn(qc, kv, c["Wo"].T, c["bo"], xq)
    f = blk["ffn"]
    g = _mm(xq2, f["W1"].T, f["b1"], act="gelu", ln=(blk["n2g"], blk["n2b"]))
    return _mm(g, f["W2"].T, f["b2"], res=xq2)


def kernel(x, params):
    p = params
    codes_p = _pool(x, p["pool"])
    qout_f, idx_f, loss11 = _rvq(codes_p.reshape(BNWP, D), p["rvq_emb"])
    indices = idx_f.reshape(B, NWP, L)[:, :NW]
    loss = loss11.reshape(())

    blk0, blk1 = p["blocks"]

    u2 = _enc_block(p["pos"], blk0["enc"], blk0["n0g"], blk0["n0b"], 1)
    xq = _cross_ffn(u2, blk0, qout_f)

    xq = _enc_block(xq, blk1["enc"], blk1["n0g"], blk1["n0b"], B)
    xq = _cross_ffn(xq, blk1, qout_f)

    return xq.reshape(B, T, D), loss, indices


# speedup vs baseline: 3.0576x; 1.0515x over previous
import functools

import jax
import jax.numpy as jnp
import numpy as np
from jax.experimental import pallas as pl

B = 4; T = 2048; D = 512; NH = 8; HD = D // NH
K = 1024; L = 4; WIN = 25; STRIDE = 12; HID = 128
NW = (T - WIN) // STRIDE + 1
NWP = 176
BNWP = B * NWP
_H = jax.lax.Precision.HIGHEST
_F = jnp.float32
_SCALE = np.float32(1.0 / np.sqrt(HD))


def _dot(a, b, dims=None):
    if dims is None:
        return jnp.dot(a, b, preferred_element_type=_F, precision=_H)
    return jax.lax.dot_general(a, b, (dims, ((), ())),
                               preferred_element_type=_F, precision=_H)


def _dot16(a, b, dims=((1,), (1,))):
    return jax.lax.dot_general(a.astype(jnp.bfloat16), b.astype(jnp.bfloat16),
                               (dims, ((), ())), preferred_element_type=_F)


def _lnf(x, g, b):
    m = jnp.mean(x, axis=1, keepdims=True)
    v = jnp.mean((x - m) ** 2, axis=1, keepdims=True)
    return (x - m) / jnp.sqrt(v + 1e-5) * g + b


def _gelu(a):
    return 0.5 * a * (1.0 + jax.lax.erf(a * np.float32(1.0 / np.sqrt(2.0))))


def _mha_heads(q2, k2, v2, mask_cols=None):
    outs = []
    for h in range(NH):
        sl = slice(h * HD, (h + 1) * HD)
        s = _dot16(q2[:, sl], k2[:, sl]) * _SCALE
        if mask_cols is not None:
            col = jax.lax.broadcasted_iota(jnp.int32, s.shape, 1)
            s = jnp.where(col < mask_cols, s, -1e30)
        m = jnp.max(s, axis=1, keepdims=True)
        p = jnp.exp(s - m)
        outs.append(_dot16(p, v2[:, sl], ((1,), (0,)))
                    / jnp.sum(p, axis=1, keepdims=True))
    return jnp.concatenate(outs, axis=1)


def _mm_kern(*refs, act, ln, res):
    x_ref, w_ref, b_ref = refs[:3]
    i = 3
    if ln:
        g_ref, be_ref = refs[i:i + 2]
        i += 2
    o_ref = refs[-1]
    x = x_ref[...]
    if ln:
        x = _lnf(x, g_ref[...], be_ref[...])
    a = _dot16(x, w_ref[...], ((1,), (0,))) + b_ref[...]
    if act == "gelu":
        a = _gelu(a)
    if res:
        a = refs[i][...] + a
    o_ref[...] = a


def _mm(x, w, b, act=None, ln=None, res=None, bm=512):
    M, Kd = x.shape
    N = w.shape[1]
    bm = min(bm, M)
    row = pl.BlockSpec((bm, Kd), lambda i: (i, 0))
    in_specs = [row,
                pl.BlockSpec((Kd, N), lambda i: (0, 0)),
                pl.BlockSpec((1, N), lambda i: (0, 0))]
    args = [x, w, b.reshape(1, N)]
    if ln is not None:
        vec = pl.BlockSpec((1, Kd), lambda i: (0, 0))
        in_specs += [vec, vec]
        args += [ln[0].reshape(1, Kd), ln[1].reshape(1, Kd)]
    if res is not None:
        nrep = res.shape[0] // bm
        in_specs.append(pl.BlockSpec((bm, N), lambda i, n=nrep: (i % n, 0)))
        args.append(res)
    return pl.pallas_call(
        functools.partial(_mm_kern, act=act, ln=ln is not None,
                          res=res is not None),
        grid=(M // bm,),
        in_specs=in_specs,
        out_specs=pl.BlockSpec((bm, N), lambda i: (i, 0)),
        out_shape=jax.ShapeDtypeStruct((M, N), _F),
    )(*args)


def _ffn_core(x, w1_ref, b1_ref, w2_ref, b2_ref):
    g = _gelu(_dot16(x, w1_ref[...], ((1,), (0,))) + b1_ref[...])
    return _dot16(g, w2_ref[...], ((1,), (0,))) + b2_ref[...]


def _effn_kern(h_ref, xq_ref, w1_ref, b1_ref, w2_ref, b2_ref, g_ref, be_ref,
               o_ref):
    ff = _ffn_core(h_ref[...], w1_ref, b1_ref, w2_ref, b2_ref)
    o_ref[...] = xq_ref[...] + _lnf(h_ref[...] + ff, g_ref[...], be_ref[...])


def _cffn_kern(xq_ref, n_g_ref, n_b_ref, w1_ref, b1_ref, w2_ref, b2_ref,
               o_ref):
    xn = _lnf(xq_ref[...], n_g_ref[...], n_b_ref[...])
    o_ref[...] = xq_ref[...] + _ffn_core(xn, w1_ref, b1_ref, w2_ref, b2_ref)


def _ffn_specs(bm):
    row = pl.BlockSpec((bm, D), lambda i: (i, 0))
    vec = pl.BlockSpec((1, D), lambda i: (0, 0))
    hvec = pl.BlockSpec((1, 4 * D), lambda i: (0, 0))
    wspec = [pl.BlockSpec((D, 4 * D), lambda i: (0, 0)), hvec,
             pl.BlockSpec((4 * D, D), lambda i: (0, 0)), vec]
    return row, vec, wspec


def _effn(h, xq, w1, b1, w2, b2, g, be, bm=512):
    M = h.shape[0]
    row, vec, wspec = _ffn_specs(bm)
    return pl.pallas_call(
        _effn_kern,
        grid=(M // bm,),
        in_specs=[row, row] + wspec + [vec, vec],
        out_specs=row,
        out_shape=jax.ShapeDtypeStruct((M, D), _F),
    )(h, xq, w1, b1.reshape(1, 4 * D), w2, b2.reshape(1, D),
      g.reshape(1, D), be.reshape(1, D))


def _cffn(xq, ng, nb_, w1, b1, w2, b2, bm=512):
    M = xq.shape[0]
    row, vec, wspec = _ffn_specs(bm)
    return pl.pallas_call(
        _cffn_kern,
        grid=(M // bm,),
        in_specs=[row, vec, vec] + wspec,
        out_specs=row,
        out_shape=jax.ShapeDtypeStruct((M, D), _F),
    )(xq, ng.reshape(1, D), nb_.reshape(1, D), w1, b1.reshape(1, 4 * D),
      w2, b2.reshape(1, D))


def _sattn_kern(q_ref, k_ref, v_ref, xq_ref, wo_ref, bo_ref,
                g0_ref, b0_ref, g1_ref, b1_ref, o_ref):
    ao = _mha_heads(q_ref[0, :, 0, 0, :], k_ref[0, :, 1, 0, :],
                    v_ref[0, :, 2, 0, :])
    a = _dot16(ao, wo_ref[...], ((1,), (0,))) + bo_ref[...]
    xn = _lnf(xq_ref[...], g0_ref[...], b0_ref[...])
    o_ref[...] = _lnf(xn + a, g1_ref[...], b1_ref[...])


def _sattn(qkv, xq, wo, bo, g0, b0, g1, b1, nb, bq=512):
    qkv5 = qkv.reshape(nb, T, 3, 1, D)
    nqt = T // bq
    row = pl.BlockSpec((bq, D), lambda b, i, n=nqt: (b * n + i, 0))
    vec = pl.BlockSpec((1, D), lambda b, i: (0, 0))
    return pl.pallas_call(
        _sattn_kern,
        grid=(nb, nqt),
        in_specs=[pl.BlockSpec((1, bq, 1, 1, D), lambda b, i: (b, i, 0, 0, 0)),
                  pl.BlockSpec((1, T, 1, 1, D), lambda b, i: (b, 0, 1, 0, 0)),
                  pl.BlockSpec((1, T, 1, 1, D), lambda b, i: (b, 0, 2, 0, 0)),
                  row, pl.BlockSpec((D, D), lambda b, i: (0, 0)), vec,
                  vec, vec, vec, vec],
        out_specs=row,
        out_shape=jax.ShapeDtypeStruct((nb * T, D), _F),
    )(qkv5, qkv5, qkv5, xq, wo, bo.reshape(1, D), g0.reshape(1, D),
      b0.reshape(1, D), g1.reshape(1, D), b1.reshape(1, D))


def _cattn_kern(q_ref, k_ref, v_ref, xq_ref, wo_ref, bo_ref, o_ref):
    ao = _mha_heads(q_ref[...], k_ref[0, :, 0, 0, :], v_ref[0, :, 1, 0, :],
                    mask_cols=NW)
    o_ref[...] = xq_ref[...] + _dot16(ao, wo_ref[...], ((1,), (0,))) + bo_ref[...]


def _cattn(qc, kv, wo, bo, xq, bq=512):
    kv5 = kv.reshape(B, NWP, 2, 1, D)
    nqt = T // bq
    nq = qc.shape[0] // bq
    nr = xq.shape[0] // bq
    qrow = pl.BlockSpec((bq, D), lambda b, i, n=nqt, m=nq: ((b * n + i) % m, 0))
    rrow = pl.BlockSpec((bq, D), lambda b, i, n=nqt, m=nr: ((b * n + i) % m, 0))
    orow = pl.BlockSpec((bq, D), lambda b, i, n=nqt: (b * n + i, 0))
    vec = pl.BlockSpec((1, D), lambda b, i: (0, 0))
    return pl.pallas_call(
        _cattn_kern,
        grid=(B, nqt),
        in_specs=[qrow,
                  pl.BlockSpec((1, NWP, 1, 1, D), lambda b, i: (b, 0, 0, 0, 0)),
                  pl.BlockSpec((1, NWP, 1, 1, D), lambda b, i: (b, 0, 1, 0, 0)),
                  rrow, pl.BlockSpec((D, D), lambda b, i: (0, 0)), vec],
        out_specs=orow,
        out_shape=jax.ShapeDtypeStruct((B * T, D), _F),
    )(qc, kv5, kv5, xq, wo, bo.reshape(1, D))


def _pool_kern(x_ref, w1t_ref, b1_ref, w2_ref, b2_ref, wot_ref, bo_ref, o_ref):
    xb = x_ref[0]
    t = jnp.tanh(_dot16(xb, w1t_ref[...], ((1,), (0,))) + b1_ref[...])
    sT = _dot16(w2_ref[...], t) + b2_ref[...]
    row = jax.lax.broadcasted_iota(jnp.int32, (NWP, T), 0)
    col = jax.lax.broadcasted_iota(jnp.int32, (NWP, T), 1)
    off = col - row * STRIDE
    valid = (off >= 0) & (off < WIN)
    S = jnp.where(valid, jnp.broadcast_to(sT, (NWP, T)), -1e30)
    m = jnp.max(S, axis=1, keepdims=True)
    p = jnp.exp(S - m)
    wts = p / jnp.sum(p, axis=1, keepdims=True)
    mean = _dot(wts, xb)
    ex2 = _dot(wts, xb * xb)
    std = jnp.sqrt(ex2 - mean * mean + 1e-6)
    cat = jnp.concatenate([mean, std], axis=1)
    o_ref[0] = _dot16(cat, wot_ref[...], ((1,), (0,))) + bo_ref[...]


def _pool(x, p):
    return pl.pallas_call(
        _pool_kern,
        grid=(B,),
        in_specs=[pl.BlockSpec((1, T, D), lambda b: (b, 0, 0)),
                  pl.BlockSpec((D, HID), lambda b: (0, 0)),
                  pl.BlockSpec((1, HID), lambda b: (0, 0)),
                  pl.BlockSpec((1, HID), lambda b: (0, 0)),
                  pl.BlockSpec((1, 1), lambda b: (0, 0)),
                  pl.BlockSpec((2 * D, D), lambda b: (0, 0)),
                  pl.BlockSpec((1, D), lambda b: (0, 0))],
        out_specs=pl.BlockSpec((1, NWP, D), lambda b: (b, 0, 0)),
        out_shape=jax.ShapeDtypeStruct((B, NWP, D), _F),
    )(x, p["W1"].T, p["b1"].reshape(1, HID), p["W2"].reshape(1, HID),
      p["b2"].reshape(1, 1), p["Wo"].T, p["bo"].reshape(1, D))


def _rvq_kern(r_ref, emb_ref, q_ref, i_ref, l_ref):
    r = r_ref[...]
    qout = jnp.zeros_like(r)
    loss = jnp.zeros((1, 1), _F)
    rowid = jax.lax.broadcasted_iota(jnp.int32, (BNWP, 1), 0)
    validrow = jax.lax.rem(rowid, NWP) < NW
    iota_k = jax.lax.broadcasted_iota(jnp.int32, (BNWP, K), 1)
    ones = jnp.ones((1, D), _F)
    idx_cols = []
    for l in range(L):
        e = emb_ref[l]
        ee = _dot(ones, e * e, ((1,), (1,)))
        rr = jnp.sum(r * r, axis=1, keepdims=True)
        re = _dot16(r, e)
        d = (rr + ee) - 2.0 * re
        dmin = jnp.min(d, axis=1, keepdims=True)
        ei = jnp.min(jnp.where(d == dmin, iota_k, K), axis=1, keepdims=True)
        oh = (iota_k == ei).astype(_F)
        q = _dot(oh, e)
        diff = q - r
        sq = jnp.where(validrow, jnp.sum(diff * diff, axis=1, keepdims=True), 0.0)
        loss = loss + 0.25 * (jnp.sum(sq, axis=0, keepdims=True)
                              / np.float32(B * NW * D))
        r = r - q
        qout = qout + q
        idx_cols.append(ei)
    q_ref[...] = qout
    i_ref[...] = jnp.concatenate(idx_cols, axis=1)
    l_ref[...] = loss


def _rvq(flat, emb):
    return pl.pallas_call(
        _rvq_kern,
        out_shape=(jax.ShapeDtypeStruct((BNWP, D), _F),
                   jax.ShapeDtypeStruct((BNWP, L), jnp.int32),
                   jax.ShapeDtypeStruct((1, 1), _F)),
    )(flat, emb)


def _enc_block(xq, e, g0, b0, nb):
    qkv = _mm(xq, e["Wqkv"].T, e["bqkv"], ln=(g0, b0))
    h = _sattn(qkv, xq, e["Wo"].T, e["bo"], g0, b0, e["g1"], e["be1"], nb)
    return _effn(h, xq, e["W1"].T, e["b1"], e["W2"].T, e["b2"],
                 e["g2"], e["be2"])


def _cross_ffn(xq, blk, qout_f):
    c = blk["cross"]
    qc = _mm(xq, c["Wqkv"][:D].T, c["bqkv"][:D], ln=(blk["n1g"], blk["n1b"]))
    kv = _mm(qout_f, c["Wqkv"][D:].T, c["bqkv"][D:], bm=BNWP)
    xq2 = _cattn(qc, kv, c["Wo"].T, c["bo"], xq)
    f = blk["ffn"]
    return _cffn(xq2, blk["n2g"], blk["n2b"], f["W1"].T, f["b1"],
                 f["W2"].T, f["b2"])


def kernel(x, params):
    p = params
    codes_p = _pool(x, p["pool"])
    qout_f, idx_f, loss11 = _rvq(codes_p.reshape(BNWP, D), p["rvq_emb"])
    indices = idx_f.reshape(B, NWP, L)[:, :NW]
    loss = loss11.reshape(())

    blk0, blk1 = p["blocks"]

    u2 = _enc_block(p["pos"], blk0["enc"], blk0["n0g"], blk0["n0b"], 1)
    xq = _cross_ffn(u2, blk0, qout_f)

    xq = _enc_block(xq, blk1["enc"], blk1["n0g"], blk1["n0b"], B)
    xq = _cross_ffn(xq, blk1, qout_f)

    return xq.reshape(B, T, D), loss, indices


# speedup vs baseline: 3.1792x; 1.0398x over previous
import functools

import jax
import jax.numpy as jnp
import numpy as np
from jax.experimental import pallas as pl

B = 4; T = 2048; D = 512; NH = 8; HD = D // NH
K = 1024; L = 4; WIN = 25; STRIDE = 12; HID = 128
NW = (T - WIN) // STRIDE + 1
NWP = 176
BNWP = B * NWP
_H = jax.lax.Precision.HIGHEST
_F = jnp.float32
_SCALE = np.float32(1.0 / np.sqrt(HD))


def _dot(a, b, dims=None):
    if dims is None:
        return jnp.dot(a, b, preferred_element_type=_F, precision=_H)
    return jax.lax.dot_general(a, b, (dims, ((), ())),
                               preferred_element_type=_F, precision=_H)


def _dot16(a, b, dims=((1,), (1,))):
    return jax.lax.dot_general(a.astype(jnp.bfloat16), b.astype(jnp.bfloat16),
                               (dims, ((), ())), preferred_element_type=_F)


def _lnf(x, g, b):
    m = jnp.mean(x, axis=1, keepdims=True)
    v = jnp.mean((x - m) ** 2, axis=1, keepdims=True)
    return (x - m) / jnp.sqrt(v + 1e-5) * g + b


def _gelu(a):
    return 0.5 * a * (1.0 + jax.lax.erf(a * np.float32(1.0 / np.sqrt(2.0))))


def _mha_heads(q2, k2, v2, mask_cols=None):
    outs = []
    for h in range(NH):
        sl = slice(h * HD, (h + 1) * HD)
        s = _dot16(q2[:, sl], k2[:, sl]) * _SCALE
        if mask_cols is not None:
            col = jax.lax.broadcasted_iota(jnp.int32, s.shape, 1)
            s = jnp.where(col < mask_cols, s, -1e30)
        m = jnp.max(s, axis=1, keepdims=True)
        p = jnp.exp(s - m)
        outs.append(_dot16(p, v2[:, sl], ((1,), (0,)))
                    / jnp.sum(p, axis=1, keepdims=True))
    return jnp.concatenate(outs, axis=1)


def _mm_kern(*refs, act, ln, res):
    x_ref, w_ref, b_ref = refs[:3]
    i = 3
    if ln:
        g_ref, be_ref = refs[i:i + 2]
        i += 2
    o_ref = refs[-1]
    x = x_ref[...]
    if ln:
        x = _lnf(x, g_ref[...], be_ref[...])
    a = _dot16(x, w_ref[...]) + b_ref[...]
    if act == "gelu":
        a = _gelu(a)
    if res:
        a = refs[i][...] + a
    o_ref[...] = a.astype(o_ref.dtype)


def _mm(x, w, b, act=None, ln=None, res=None, bm=512, out_dtype=_F):
    M, Kd = x.shape
    N = w.shape[0]
    bm = min(bm, M)
    row = pl.BlockSpec((bm, Kd), lambda i: (i, 0))
    in_specs = [row,
                pl.BlockSpec((N, Kd), lambda i: (0, 0)),
                pl.BlockSpec((1, N), lambda i: (0, 0))]
    args = [x, w, b.reshape(1, N)]
    if ln is not None:
        vec = pl.BlockSpec((1, Kd), lambda i: (0, 0))
        in_specs += [vec, vec]
        args += [ln[0].reshape(1, Kd), ln[1].reshape(1, Kd)]
    if res is not None:
        nrep = res.shape[0] // bm
        in_specs.append(pl.BlockSpec((bm, N), lambda i, n=nrep: (i % n, 0)))
        args.append(res)
    return pl.pallas_call(
        functools.partial(_mm_kern, act=act, ln=ln is not None,
                          res=res is not None),
        grid=(M // bm,),
        in_specs=in_specs,
        out_specs=pl.BlockSpec((bm, N), lambda i: (i, 0)),
        out_shape=jax.ShapeDtypeStruct((M, N), out_dtype),
    )(*args)


def _ffn_core(x, w1_ref, b1_ref, w2_ref, b2_ref):
    g = _gelu(_dot16(x, w1_ref[...]) + b1_ref[...])
    return _dot16(g, w2_ref[...]) + b2_ref[...]


def _effn_kern(h_ref, xq_ref, w1_ref, b1_ref, w2_ref, b2_ref, g_ref, be_ref,
               o_ref):
    ff = _ffn_core(h_ref[...], w1_ref, b1_ref, w2_ref, b2_ref)
    o_ref[...] = xq_ref[...] + _lnf(h_ref[...] + ff, g_ref[...], be_ref[...])


def _cffn_kern(xq_ref, n_g_ref, n_b_ref, w1_ref, b1_ref, w2_ref, b2_ref,
               o_ref):
    xn = _lnf(xq_ref[...], n_g_ref[...], n_b_ref[...])
    o_ref[...] = xq_ref[...] + _ffn_core(xn, w1_ref, b1_ref, w2_ref, b2_ref)


def _ffn_specs(bm):
    row = pl.BlockSpec((bm, D), lambda i: (i, 0))
    vec = pl.BlockSpec((1, D), lambda i: (0, 0))
    hvec = pl.BlockSpec((1, 4 * D), lambda i: (0, 0))
    wspec = [pl.BlockSpec((4 * D, D), lambda i: (0, 0)), hvec,
             pl.BlockSpec((D, 4 * D), lambda i: (0, 0)), vec]
    return row, vec, wspec


def _effn(h, xq, w1, b1, w2, b2, g, be, bm=512):
    M = h.shape[0]
    row, vec, wspec = _ffn_specs(bm)
    return pl.pallas_call(
        _effn_kern,
        grid=(M // bm,),
        in_specs=[row, row] + wspec + [vec, vec],
        out_specs=row,
        out_shape=jax.ShapeDtypeStruct((M, D), _F),
    )(h, xq, w1, b1.reshape(1, 4 * D), w2, b2.reshape(1, D),
      g.reshape(1, D), be.reshape(1, D))


def _cffn(xq, ng, nb_, w1, b1, w2, b2, bm=512):
    M = xq.shape[0]
    row, vec, wspec = _ffn_specs(bm)
    return pl.pallas_call(
        _cffn_kern,
        grid=(M // bm,),
        in_specs=[row, vec, vec] + wspec,
        out_specs=row,
        out_shape=jax.ShapeDtypeStruct((M, D), _F),
    )(xq, ng.reshape(1, D), nb_.reshape(1, D), w1, b1.reshape(1, 4 * D),
      w2, b2.reshape(1, D))


def _sattn_kern(q_ref, k_ref, v_ref, xq_ref, wo_ref, bo_ref,
                g0_ref, b0_ref, g1_ref, b1_ref, o_ref):
    ao = _mha_heads(q_ref[0, :, 0, 0, :], k_ref[0, :, 1, 0, :],
                    v_ref[0, :, 2, 0, :])
    a = _dot16(ao, wo_ref[...]) + bo_ref[...]
    xn = _lnf(xq_ref[...], g0_ref[...], b0_ref[...])
    o_ref[...] = _lnf(xn + a, g1_ref[...], b1_ref[...])


def _sattn(qkv, xq, wo, bo, g0, b0, g1, b1, nb, bq=512):
    qkv5 = qkv.reshape(nb, T, 3, 1, D)
    nqt = T // bq
    row = pl.BlockSpec((bq, D), lambda b, i, n=nqt: (b * n + i, 0))
    vec = pl.BlockSpec((1, D), lambda b, i: (0, 0))
    return pl.pallas_call(
        _sattn_kern,
        grid=(nb, nqt),
        in_specs=[pl.BlockSpec((1, bq, 1, 1, D), lambda b, i: (b, i, 0, 0, 0)),
                  pl.BlockSpec((1, T, 1, 1, D), lambda b, i: (b, 0, 1, 0, 0)),
                  pl.BlockSpec((1, T, 1, 1, D), lambda b, i: (b, 0, 2, 0, 0)),
                  row, pl.BlockSpec((D, D), lambda b, i: (0, 0)), vec,
                  vec, vec, vec, vec],
        out_specs=row,
        out_shape=jax.ShapeDtypeStruct((nb * T, D), _F),
    )(qkv5, qkv5, qkv5, xq, wo, bo.reshape(1, D), g0.reshape(1, D),
      b0.reshape(1, D), g1.reshape(1, D), b1.reshape(1, D))


def _cattn_kern(q_ref, k_ref, v_ref, xq_ref, wo_ref, bo_ref, o_ref):
    ao = _mha_heads(q_ref[...], k_ref[0, :, 0, 0, :], v_ref[0, :, 1, 0, :],
                    mask_cols=NW)
    o_ref[...] = xq_ref[...] + _dot16(ao, wo_ref[...]) + bo_ref[...]


def _cattn(qc, kv, wo, bo, xq, bq=512):
    kv5 = kv.reshape(B, NWP, 2, 1, D)
    nqt = T // bq
    nq = qc.shape[0] // bq
    nr = xq.shape[0] // bq
    qrow = pl.BlockSpec((bq, D), lambda b, i, n=nqt, m=nq: ((b * n + i) % m, 0))
    rrow = pl.BlockSpec((bq, D), lambda b, i, n=nqt, m=nr: ((b * n + i) % m, 0))
    orow = pl.BlockSpec((bq, D), lambda b, i, n=nqt: (b * n + i, 0))
    vec = pl.BlockSpec((1, D), lambda b, i: (0, 0))
    return pl.pallas_call(
        _cattn_kern,
        grid=(B, nqt),
        in_specs=[qrow,
                  pl.BlockSpec((1, NWP, 1, 1, D), lambda b, i: (b, 0, 0, 0, 0)),
                  pl.BlockSpec((1, NWP, 1, 1, D), lambda b, i: (b, 0, 1, 0, 0)),
                  rrow, pl.BlockSpec((D, D), lambda b, i: (0, 0)), vec],
        out_specs=orow,
        out_shape=jax.ShapeDtypeStruct((B * T, D), _F),
    )(qc, kv5, kv5, xq, wo, bo.reshape(1, D))


def _pool_kern(x_ref, w1t_ref, b1_ref, w2_ref, b2_ref, wot_ref, bo_ref, o_ref):
    xb = x_ref[0]
    t = jnp.tanh(_dot16(xb, w1t_ref[...]) + b1_ref[...])
    sT = _dot16(w2_ref[...], t) + b2_ref[...]
    row = jax.lax.broadcasted_iota(jnp.int32, (NWP, T), 0)
    col = jax.lax.broadcasted_iota(jnp.int32, (NWP, T), 1)
    off = col - row * STRIDE
    valid = (off >= 0) & (off < WIN)
    S = jnp.where(valid, jnp.broadcast_to(sT, (NWP, T)), -1e30)
    m = jnp.max(S, axis=1, keepdims=True)
    p = jnp.exp(S - m)
    wts = p / jnp.sum(p, axis=1, keepdims=True)
    mean = _dot(wts, xb)
    ex2 = _dot(wts, xb * xb)
    std = jnp.sqrt(ex2 - mean * mean + 1e-6)
    cat = jnp.concatenate([mean, std], axis=1)
    o_ref[0] = _dot16(cat, wot_ref[...]) + bo_ref[...]


def _pool(x, p):
    return pl.pallas_call(
        _pool_kern,
        grid=(B,),
        in_specs=[pl.BlockSpec((1, T, D), lambda b: (b, 0, 0)),
                  pl.BlockSpec((HID, D), lambda b: (0, 0)),
                  pl.BlockSpec((1, HID), lambda b: (0, 0)),
                  pl.BlockSpec((1, HID), lambda b: (0, 0)),
                  pl.BlockSpec((1, 1), lambda b: (0, 0)),
                  pl.BlockSpec((D, 2 * D), lambda b: (0, 0)),
                  pl.BlockSpec((1, D), lambda b: (0, 0))],
        out_specs=pl.BlockSpec((1, NWP, D), lambda b: (b, 0, 0)),
        out_shape=jax.ShapeDtypeStruct((B, NWP, D), _F),
    )(x, p["W1"], p["b1"].reshape(1, HID), p["W2"].reshape(1, HID),
      p["b2"].reshape(1, 1), p["Wo"], p["bo"].reshape(1, D))


def _rvq_kern(r_ref, emb_ref, q_ref, i_ref, l_ref):
    r = r_ref[...]
    qout = jnp.zeros_like(r)
    loss = jnp.zeros((1, 1), _F)
    rowid = jax.lax.broadcasted_iota(jnp.int32, (BNWP, 1), 0)
    validrow = jax.lax.rem(rowid, NWP) < NW
    iota_k = jax.lax.broadcasted_iota(jnp.int32, (BNWP, K), 1)
    ones = jnp.ones((1, D), _F)
    idx_cols = []
    for l in range(L):
        e = emb_ref[l]
        ee = _dot(ones, e * e, ((1,), (1,)))
        rr = jnp.sum(r * r, axis=1, keepdims=True)
        re = _dot16(r, e)
        d = (rr + ee) - 2.0 * re
        dmin = jnp.min(d, axis=1, keepdims=True)
        ei = jnp.min(jnp.where(d == dmin, iota_k, K), axis=1, keepdims=True)
        oh = (iota_k == ei).astype(_F)
        q = _dot(oh, e)
        diff = q - r
        sq = jnp.where(validrow, jnp.sum(diff * diff, axis=1, keepdims=True), 0.0)
        loss = loss + 0.25 * (jnp.sum(sq, axis=0, keepdims=True)
                              / np.float32(B * NW * D))
        r = r - q
        qout = qout + q
        idx_cols.append(ei)
    q_ref[...] = qout
    i_ref[...] = jnp.concatenate(idx_cols, axis=1)
    l_ref[...] = loss


def _rvq(flat, emb):
    return pl.pallas_call(
        _rvq_kern,
        out_shape=(jax.ShapeDtypeStruct((BNWP, D), _F),
                   jax.ShapeDtypeStruct((BNWP, L), jnp.int32),
                   jax.ShapeDtypeStruct((1, 1), _F)),
    )(flat, emb)


def _enc_block(xq, e, g0, b0, nb):
    qkv = _mm(xq, e["Wqkv"], e["bqkv"], ln=(g0, b0))
    h = _sattn(qkv, xq, e["Wo"], e["bo"], g0, b0, e["g1"], e["be1"], nb)
    return _effn(h, xq, e["W1"], e["b1"], e["W2"], e["b2"],
                 e["g2"], e["be2"])


def _cross_ffn(xq, blk, qout_f):
    c = blk["cross"]
    qc = _mm(xq, c["Wqkv"][:D], c["bqkv"][:D], ln=(blk["n1g"], blk["n1b"]))
    kv = _mm(qout_f, c["Wqkv"][D:], c["bqkv"][D:], bm=BNWP)
    xq2 = _cattn(qc, kv, c["Wo"], c["bo"], xq)
    f = blk["ffn"]
    return _cffn(xq2, blk["n2g"], blk["n2b"], f["W1"], f["b1"],
                 f["W2"], f["b2"])


def kernel(x, params):
    p = params
    codes_p = _pool(x, p["pool"])
    qout_f, idx_f, loss11 = _rvq(codes_p.reshape(BNWP, D), p["rvq_emb"])
    indices = idx_f.reshape(B, NWP, L)[:, :NW]
    loss = loss11.reshape(())

    blk0, blk1 = p["blocks"]

    u2 = _enc_block(p["pos"], blk0["enc"], blk0["n0g"], blk0["n0b"], 1)
    xq = _cross_ffn(u2, blk0, qout_f)

    xq = _enc_block(xq, blk1["enc"], blk1["n0g"], blk1["n0b"], B)
    xq = _cross_ffn(xq, blk1, qout_f)

    return xq.reshape(B, T, D), loss, indices
